# baseline (speedup 1.0000x reference)
import functools

import jax
import jax.numpy as jnp
from jax import lax
from jax.experimental import pallas as pl
from jax.experimental.pallas import tpu as pltpu
from jax.experimental.pallas import tpu_sc as plsc

HID = 64
BS, N, K, M, KO = 8, 1024, 32, 256, 16
BN = BS * N
BLK = 256
NW = 32
GC = 128


def _silu(x):
    return x * jax.nn.sigmoid(x)


def _sc_gather_rows(table, idx, D):
    E = idx.shape[0]
    per = E // NW
    nch = per // GC
    idx3 = idx.reshape(NW, nch, GC)
    mesh = plsc.VectorSubcoreMesh(core_axis_name="c", subcore_axis_name="s")

    @functools.partial(
        pl.kernel, mesh=mesh,
        compiler_params=pltpu.CompilerParams(use_tc_tiling_on_sc=False),
        out_type=jax.ShapeDtypeStruct((E, D), jnp.float32),
        scratch_types=[
            pltpu.VMEM((nch, GC), jnp.int32),
            pltpu.VMEM((2, GC, D), jnp.float32),
            pltpu.SemaphoreType.DMA,
            pltpu.SemaphoreType.DMA,
            pltpu.SemaphoreType.DMA,
        ],
    )
    def k(table_hbm, idx_hbm, out_hbm, idx_v, rows_v, gsem, ssem0, ssem1):
        wid = lax.axis_index("s") * 2 + lax.axis_index("c")
        base = wid * per
        pltpu.sync_copy(idx_hbm.at[wid], idx_v)
        ssems = [ssem0, ssem1]
        stores = [None, None]
        for g in range(nch):
            b = g % 2
            if stores[b] is not None:
                stores[b].wait()
            pltpu.async_copy(table_hbm.at[idx_v.at[g]], rows_v.at[b], gsem).wait()
            stores[b] = pltpu.async_copy(
                rows_v.at[b], out_hbm.at[pl.ds(base + g * GC, GC)], ssems[b])
        for s in stores:
            if s is not None:
                s.wait()

    return k(table, idx3)


def _sc_dxy(xt, yt, jloc, iloc, maskf, rpb, epb):
    E = jloc.shape[0]
    per = E // NW
    nv = per // 16
    mesh = plsc.VectorSubcoreMesh(core_axis_name="c", subcore_axis_name="s")

    @functools.partial(
        pl.kernel, mesh=mesh,
        compiler_params=pltpu.CompilerParams(needs_layout_passes=False),
        out_type=(jax.ShapeDtypeStruct((E,), jnp.float32),
                  jax.ShapeDtypeStruct((E,), jnp.float32)),
        scratch_types=[
            pltpu.VMEM((rpb,), jnp.float32),
            pltpu.VMEM((rpb,), jnp.float32),
            pltpu.VMEM((per,), jnp.int32),
            pltpu.VMEM((per,), jnp.int32),
            pltpu.VMEM((per,), jnp.float32),
            pltpu.VMEM((per,), jnp.float32),
            pltpu.VMEM((per,), jnp.float32),
        ],
    )
    def k(xt_hbm, yt_hbm, j_hbm, i_hbm, m_hbm, ox_hbm, oy_hbm,
          xv, yv, jv, iv, mv, dxv, dyv):
        wid = lax.axis_index("s") * 2 + lax.axis_index("c")
        base = wid * per
        b = (wid * per) // epb
        pltpu.sync_copy(xt_hbm.at[pl.ds(b * rpb, rpb)], xv)
        pltpu.sync_copy(yt_hbm.at[pl.ds(b * rpb, rpb)], yv)
        pltpu.sync_copy(j_hbm.at[pl.ds(base, per)], jv)
        pltpu.sync_copy(i_hbm.at[pl.ds(base, per)], iv)
        pltpu.sync_copy(m_hbm.at[pl.ds(base, per)], mv)

        def body(v, c):
            off = pl.multiple_of(v * 16, 16)
            j = jv[pl.ds(off, 16)]
            i = iv[pl.ds(off, 16)]
            mk = mv[pl.ds(off, 16)]
            dxv[pl.ds(off, 16)] = (
                plsc.load_gather(xv, [j]) - plsc.load_gather(xv, [i])) * mk
            dyv[pl.ds(off, 16)] = (
                plsc.load_gather(yv, [j]) - plsc.load_gather(yv, [i])) * mk
            return c

        lax.fori_loop(0, nv, body, 0)
        pltpu.sync_copy(dxv, ox_hbm.at[pl.ds(base, per)])
        pltpu.sync_copy(dyv, oy_hbm.at[pl.ds(base, per)])

    return k(xt, yt, jloc, iloc, maskf)


def _sc_posj(xt, yt, jloc, rpb, epb):
    E = jloc.shape[0]
    per = E // NW
    nv = per // 16
    mesh = plsc.VectorSubcoreMesh(core_axis_name="c", subcore_axis_name="s")

    @functools.partial(
        pl.kernel, mesh=mesh,
        compiler_params=pltpu.CompilerParams(needs_layout_passes=False),
        out_type=(jax.ShapeDtypeStruct((E,), jnp.float32),
                  jax.ShapeDtypeStruct((E,), jnp.float32)),
        scratch_types=[
            pltpu.VMEM((rpb,), jnp.float32),
            pltpu.VMEM((rpb,), jnp.float32),
            pltpu.VMEM((per,), jnp.int32),
            pltpu.VMEM((per,), jnp.float32),
            pltpu.VMEM((per,), jnp.float32),
        ],
    )
    def k(xt_hbm, yt_hbm, j_hbm, ox_hbm, oy_hbm, xv, yv, jv, pxv, pyv):
        wid = lax.axis_index("s") * 2 + lax.axis_index("c")
        base = wid * per
        b = (wid * per) // epb
        pltpu.sync_copy(xt_hbm.at[pl.ds(b * rpb, rpb)], xv)
        pltpu.sync_copy(yt_hbm.at[pl.ds(b * rpb, rpb)], yv)
        pltpu.sync_copy(j_hbm.at[pl.ds(base, per)], jv)

        def body(v, c):
            off = pl.multiple_of(v * 16, 16)
            j = jv[pl.ds(off, 16)]
            pxv[pl.ds(off, 16)] = plsc.load_gather(xv, [j])
            pyv[pl.ds(off, 16)] = plsc.load_gather(yv, [j])
            return c

        lax.fori_loop(0, nv, body, 0)
        pltpu.sync_copy(pxv, ox_hbm.at[pl.ds(base, per)])
        pltpu.sync_copy(pyv, oy_hbm.at[pl.ds(base, per)])

    return k(xt, yt, jloc)


def _rep(x, k):
    blk, w = x.shape
    return jnp.broadcast_to(x[:, None, :], (blk, k, w)).reshape(blk * k, w)


def _embed_body(pf_ref, ke_ref, evW_ref, evb_ref, eaW_ref, eab_ref,
                embW_ref, embb_ref, w1b_ref, h_ref, b_ref):
    pf = pf_ref[...]
    embW = embW_ref[...]
    spd = jnp.sqrt(pf[:, 2:3] ** 2 + pf[:, 3:4] ** 2)
    acc = jnp.sqrt(pf[:, 4:5] ** 2 + pf[:, 5:6] ** 2)
    cv = evW_ref[...] @ embW[0:8]
    ca = eaW_ref[...] @ embW[8:16]
    c0 = evb_ref[...] @ embW[0:8] + eab_ref[...] @ embW[8:16] + embb_ref[...]
    h = spd * cv + acc * ca + ke_ref[...] @ embW[16:19] + c0
    h_ref[...] = h
    b_ref[...] = h @ w1b_ref[...]


def _embed_call(pf, ke, pr):
    grid = (BN // BLK,)
    full = lambda s: pl.BlockSpec(s, lambda g: (0, 0))
    return pl.pallas_call(
        _embed_body,
        grid=grid,
        in_specs=[
            pl.BlockSpec((BLK, 6), lambda g: (g, 0)),
            pl.BlockSpec((BLK, 3), lambda g: (g, 0)),
            full((1, 8)), full((1, 8)), full((1, 8)), full((1, 8)),
            full((19, HID)), full((1, HID)), full((HID, HID)),
        ],
        out_specs=[
            pl.BlockSpec((BLK, HID), lambda g: (g, 0)),
            pl.BlockSpec((BLK, HID), lambda g: (g, 0)),
        ],
        out_shape=[
            jax.ShapeDtypeStruct((BN, HID), jnp.float32),
            jax.ShapeDtypeStruct((BN, HID), jnp.float32),
        ],
    )(pf, ke,
      pr["enc_v_W"].reshape(1, 8), pr["enc_v_b"].reshape(1, 8),
      pr["enc_a_W"].reshape(1, 8), pr["enc_a_b"].reshape(1, 8),
      pr["emb_W"], pr["emb_b"].reshape(1, HID),
      pr["gnn0_W1b"])


def _edge_math(h, st, Bj, dxm, dym, mask128, maskD, w, kk):
    eb = BLK * kk
    r = eb // 128
    d = jnp.sqrt(dxm * dxm + dym * dym)
    A = h @ w["W1a"] + w["feb1"]
    base = _rep(A, kk)
    if Bj is not None:
        base = base + Bj
    pre1 = base.reshape(r, 128, HID) + d[:, :, None] * w["w1d"].reshape(1, 1, HID)
    u = _silu(pre1).reshape(eb, HID)
    m3 = (_silu(u @ w["feW2"] + w["feb2"]).reshape(r, 128, HID)
          * mask128[:, :, None])
    m = m3.reshape(eb, HID)
    t = _silu(m @ w["fxW1"] + w["fxb1"])
    s = jnp.sum(t.reshape(r, 128, HID) * w["fxW2r"].reshape(1, 1, HID),
                axis=2) + w["fxb2"]
    r3 = jnp.concatenate(
        [m3, (dxm * s)[:, :, None], (dym * s)[:, :, None]], axis=2)
    rr = jnp.sum(r3.reshape(BLK, kk, HID + 2), axis=1)
    m_i = rr[:, 0:HID]
    aggx = rr[:, HID:HID + 1]
    aggy = rr[:, HID + 1:HID + 2]
    nn = jnp.sum(maskD, axis=1, keepdims=True)
    inv = 1.0 / (nn + 1e-6)
    fa = _silu(h @ w["faW1"] + w["fab1"]) @ w["faW2c"] + w["fab2"]
    ax = fa * st[:, 4:5] + aggx * inv
    ay = fa * st[:, 5:6] + aggy * inv
    vx = st[:, 2:3] + ax
    vy = st[:, 3:4] + ay
    xx = st[:, 0:1] + vx
    xy = st[:, 1:2] + vy
    h_new = h + _silu(h @ w["fhW1h"] + m_i @ w["fhW1m"] + w["fhb1"]) @ w["fhW2"] + w["fhb2"]
    st_new = jnp.concatenate([xx, xy, vx, vy, st[:, 4:8]], axis=1)
    return h_new, st_new


_WNAMES = ["W1a", "w1d", "feb1", "feW2", "feb2", "fxW1", "fxb1", "fxW2r",
           "fxb2", "faW1", "fab1", "faW2c", "fab2", "fhW1h", "fhW1m",
           "fhb1", "fhW2", "fhb2"]


def _edge_body(h_ref, st_ref, bj_ref, dx_ref, dy_ref, mp_ref, md_ref, *w_refs):
    (w1bn_ref, *wl), outs = (w_refs[:-5], w_refs[-5:])
    hn_ref, stn_ref, bn_ref, px_ref, py_ref = outs
    w = {n: r[...] for n, r in zip(_WNAMES, wl)}
    h = h_ref[...]
    st = st_ref[...]
    h_new, st_new = _edge_math(h, st, bj_ref[...], dx_ref[...], dy_ref[...],
                               mp_ref[...], md_ref[...], w, K)
    hn_ref[...] = h_new
    stn_ref[...] = st_new
    bn_ref[...] = h_new @ w1bn_ref[...]
    px_ref[...] = st_new[:, 0:1]
    py_ref[...] = st_new[:, 1:2]


def _edge_call(h, st, bj, dxm, dym, maskP, maskD, wl, w1bn):
    grid = (BN // BLK,)
    full = lambda s: pl.BlockSpec(s, lambda g: tuple(0 for _ in s))
    wspecs = [full(w1bn.shape)] + [full(wl[n].shape) for n in _WNAMES]
    rp = (BLK * K) // 128
    return pl.pallas_call(
        _edge_body,
        grid=grid,
        in_specs=[
            pl.BlockSpec((BLK, HID), lambda g: (g, 0)),
            pl.BlockSpec((BLK, 8), lambda g: (g, 0)),
            pl.BlockSpec((BLK * K, HID), lambda g: (g, 0)),
            pl.BlockSpec((rp, 128), lambda g: (g, 0)),
            pl.BlockSpec((rp, 128), lambda g: (g, 0)),
            pl.BlockSpec((rp, 128), lambda g: (g, 0)),
            pl.BlockSpec((BLK, K), lambda g: (g, 0)),
        ] + wspecs,
        out_specs=[
            pl.BlockSpec((BLK, HID), lambda g: (g, 0)),
            pl.BlockSpec((BLK, 8), lambda g: (g, 0)),
            pl.BlockSpec((BLK, HID), lambda g: (g, 0)),
            pl.BlockSpec((BLK, 1), lambda g: (g, 0)),
            pl.BlockSpec((BLK, 1), lambda g: (g, 0)),
        ],
        out_shape=[
            jax.ShapeDtypeStruct((BN, HID), jnp.float32),
            jax.ShapeDtypeStruct((BN, 8), jnp.float32),
            jax.ShapeDtypeStruct((BN, HID), jnp.float32),
            jax.ShapeDtypeStruct((BN, 1), jnp.float32),
            jax.ShapeDtypeStruct((BN, 1), jnp.float32),
        ],
    )(h, st, bj, dxm, dym, maskP, maskD, w1bn, *[wl[n] for n in _WNAMES])


def _obs_body(h_ref, st_ref, ox_ref, oy_ref, mp_ref, md_ref, *w_refs):
    wl3, (hn_ref,) = w_refs[:-1], w_refs[-1:]
    h = h_ref[...]
    st = st_ref[...]
    rp = (BLK * KO) // 128
    mask128 = mp_ref[...]
    maskD = md_ref[...]
    opx = ox_ref[...]
    opy = oy_ref[...]
    for l in range(3):
        w = {n: r[l] for n, r in zip(_WNAMES, wl3)}
        pix = _rep(st[:, 0:1], KO).reshape(rp, 128)
        piy = _rep(st[:, 1:2], KO).reshape(rp, 128)
        dxm = (opx - pix) * mask128
        dym = (opy - piy) * mask128
        h, st = _edge_math(h, st, None, dxm, dym, mask128, maskD, w, KO)
    hn_ref[...] = h


def _obs_call(h, st, opx, opy, maskP, maskD, wl3):
    grid = (BN // BLK,)
    full = lambda s: pl.BlockSpec(s, lambda g: tuple(0 for _ in s))
    wspecs = [full(wl3[n].shape) for n in _WNAMES]
    rp = (BLK * KO) // 128
    return pl.pallas_call(
        _obs_body,
        grid=grid,
        in_specs=[
            pl.BlockSpec((BLK, HID), lambda g: (g, 0)),
            pl.BlockSpec((BLK, 8), lambda g: (g, 0)),
            pl.BlockSpec((rp, 128), lambda g: (g, 0)),
            pl.BlockSpec((rp, 128), lambda g: (g, 0)),
            pl.BlockSpec((rp, 128), lambda g: (g, 0)),
            pl.BlockSpec((BLK, KO), lambda g: (g, 0)),
        ] + wspecs,
        out_specs=pl.BlockSpec((BLK, HID), lambda g: (g, 0)),
        out_shape=jax.ShapeDtypeStruct((BN, HID), jnp.float32),
    )(h, st, opx, opy, maskP, maskD, *[wl3[n] for n in _WNAMES])


def _prep_layer(p, has_neigh):
    W1 = p["fe_W1"]
    w = {
        "W1a": W1[:HID],
        "w1d": W1[-1:],
        "feb1": p["fe_b1"].reshape(1, HID),
        "feW2": p["fe_W2"],
        "feb2": p["fe_b2"].reshape(1, HID),
        "fxW1": p["fx_W1"],
        "fxb1": p["fx_b1"].reshape(1, HID),
        "fxW2r": p["fx_W2"].reshape(1, HID),
        "fxb2": p["fx_b2"].reshape(1, 1),
        "faW1": p["fa_W1"],
        "fab1": p["fa_b1"].reshape(1, HID),
        "faW2c": p["fa_W2"].reshape(HID, 1),
        "fab2": p["fa_b2"].reshape(1, 1),
        "fhW1h": p["fh_W1"][:HID],
        "fhW1m": p["fh_W1"][HID:2 * HID],
        "fhb1": p["fh_b1"].reshape(1, HID),
        "fhW2": p["fh_W2"],
        "fhb2": p["fh_b2"].reshape(1, HID),
    }
    if has_neigh:
        w["W1b"] = W1[HID:2 * HID]
    return w


def kernel(ped_features, neigh_mask, neigh_index, obs_features,
           neigh_mask_obs, neigh_index_obs, k_emb, params):
    pf = ped_features.reshape(BN, 6)
    ke = k_emb.reshape(BN, 3)
    st0 = jnp.pad(pf, ((0, 0), (0, 2)))

    offs = (jnp.arange(BS, dtype=jnp.int32) * N)[:, None, None]
    gidx = (neigh_index.astype(jnp.int32) + offs).reshape(BN * K)
    jloc = neigh_index.astype(jnp.int32).reshape(BN * K)
    iloc = jnp.repeat(jnp.arange(BN, dtype=jnp.int32) % N, K)
    ojloc = neigh_index_obs.astype(jnp.int32).reshape(BN * KO)
    maskf = neigh_mask.astype(jnp.float32)
    maskE = maskf.reshape(BN * K)
    maskP = maskf.reshape(BN * K // 128, 128)
    maskD = maskf.reshape(BN, K)
    maskfo = neigh_mask_obs.astype(jnp.float32)
    maskPo = maskfo.reshape(BN * KO // 128, 128)
    maskDo = maskfo.reshape(BN, KO)

    gl = [_prep_layer(p, True) for p in params["gnn"]]
    ol = [_prep_layer(p, False) for p in params["obs"]]
    ol3 = {n: jnp.stack([w[n] for w in ol]) for n in _WNAMES}

    pr = dict(params)
    pr["gnn0_W1b"] = gl[0]["W1b"]
    h1, B1 = _embed_call(pf, ke, pr)

    obs_x = obs_features[..., 0].reshape(BS * M)
    obs_y = obs_features[..., 1].reshape(BS * M)
    opx, opy = _sc_posj(obs_x, obs_y, ojloc, M, N * KO)
    opx = opx.reshape(BN * KO // 128, 128)
    opy = opy.reshape(BN * KO // 128, 128)

    h_obs = _obs_call(h1, st0, opx, opy, maskPo, maskDo, ol3)

    h, st, B = h1, st0, B1
    px = pf[:, 0:1]
    py = pf[:, 1:2]
    zW = jnp.zeros((HID, HID), jnp.float32)
    for li in range(3):
        dxm, dym = _sc_dxy(px.reshape(BN), py.reshape(BN), jloc, iloc,
                           maskE, N, N * K)
        bj = _sc_gather_rows(B, gidx, HID)
        w1bn = gl[li + 1]["W1b"] if li < 2 else zW
        h, st, B, px, py = _edge_call(
            h, st, bj,
            dxm.reshape(BN * K // 128, 128), dym.reshape(BN * K // 128, 128),
            maskP, maskD, gl[li], w1bn)

    return (h.reshape(BS, N, HID), h_obs.reshape(BS, N, HID))


# speedup vs baseline: 1.0243x; 1.0243x over previous
import functools

import jax
import jax.numpy as jnp
from jax import lax
from jax.experimental import pallas as pl
from jax.experimental.pallas import tpu as pltpu
from jax.experimental.pallas import tpu_sc as plsc

HID = 64
BS, N, K, M, KO = 8, 1024, 32, 256, 16
BN = BS * N
BLK = 256
NW = 32
GC = 128


def _silu(x):
    return x * jax.nn.sigmoid(x)


def _sc_gather_rows(table, idx, D):
    E = idx.shape[0]
    per = E // NW
    nch = per // GC
    idx3 = idx.reshape(NW, nch, GC)
    mesh = plsc.VectorSubcoreMesh(core_axis_name="c", subcore_axis_name="s")

    @functools.partial(
        pl.kernel, mesh=mesh,
        compiler_params=pltpu.CompilerParams(use_tc_tiling_on_sc=False),
        out_type=jax.ShapeDtypeStruct((E, D), jnp.float32),
        scratch_types=[
            pltpu.VMEM((nch, GC), jnp.int32),
            pltpu.VMEM((4, GC, D), jnp.float32),
            pltpu.SemaphoreType.DMA,
            pltpu.SemaphoreType.DMA,
            pltpu.SemaphoreType.DMA,
            pltpu.SemaphoreType.DMA,
            pltpu.SemaphoreType.DMA,
            pltpu.SemaphoreType.DMA,
        ],
    )
    def k(table_hbm, idx_hbm, out_hbm, idx_v, rows_v,
          g0, g1, s0, s1, s2, s3):
        wid = lax.axis_index("s") * 2 + lax.axis_index("c")
        base = wid * per
        pltpu.sync_copy(idx_hbm.at[wid], idx_v)
        gsems = [g0, g1]
        ssems = [s0, s1, s2, s3]
        gathers = [None] * 4
        stores = [None] * 4
        for g in range(nch + 1):
            if g < nch:
                b = g % 4
                if stores[b] is not None:
                    stores[b].wait()
                gathers[b] = pltpu.async_copy(
                    table_hbm.at[idx_v.at[g]], rows_v.at[b], gsems[g % 2])
            if g >= 1:
                pb = (g - 1) % 4
                gathers[pb].wait()
                stores[pb] = pltpu.async_copy(
                    rows_v.at[pb],
                    out_hbm.at[pl.ds(base + (g - 1) * GC, GC)], ssems[pb])
        for s in stores:
            if s is not None:
                s.wait()

    return k(table, idx3)


def _sc_dxy(xt, yt, jloc, iloc, maskf, rpb, epb):
    E = jloc.shape[0]
    per = E // NW
    nv = per // 16
    mesh = plsc.VectorSubcoreMesh(core_axis_name="c", subcore_axis_name="s")

    @functools.partial(
        pl.kernel, mesh=mesh,
        compiler_params=pltpu.CompilerParams(needs_layout_passes=False),
        out_type=(jax.ShapeDtypeStruct((E,), jnp.float32),
                  jax.ShapeDtypeStruct((E,), jnp.float32)),
        scratch_types=[
            pltpu.VMEM((rpb,), jnp.float32),
            pltpu.VMEM((rpb,), jnp.float32),
            pltpu.VMEM((per,), jnp.int32),
            pltpu.VMEM((per,), jnp.int32),
            pltpu.VMEM((per,), jnp.float32),
            pltpu.VMEM((per,), jnp.float32),
            pltpu.VMEM((per,), jnp.float32),
        ],
    )
    def k(xt_hbm, yt_hbm, j_hbm, i_hbm, m_hbm, ox_hbm, oy_hbm,
          xv, yv, jv, iv, mv, dxv, dyv):
        wid = lax.axis_index("s") * 2 + lax.axis_index("c")
        base = wid * per
        b = (wid * per) // epb
        pltpu.sync_copy(xt_hbm.at[pl.ds(b * rpb, rpb)], xv)
        pltpu.sync_copy(yt_hbm.at[pl.ds(b * rpb, rpb)], yv)
        pltpu.sync_copy(j_hbm.at[pl.ds(base, per)], jv)
        pltpu.sync_copy(i_hbm.at[pl.ds(base, per)], iv)
        pltpu.sync_copy(m_hbm.at[pl.ds(base, per)], mv)

        def body(v, c):
            off = pl.multiple_of(v * 16, 16)
            j = jv[pl.ds(off, 16)]
            i = iv[pl.ds(off, 16)]
            mk = mv[pl.ds(off, 16)]
            dxv[pl.ds(off, 16)] = (
                plsc.load_gather(xv, [j]) - plsc.load_gather(xv, [i])) * mk
            dyv[pl.ds(off, 16)] = (
                plsc.load_gather(yv, [j]) - plsc.load_gather(yv, [i])) * mk
            return c

        lax.fori_loop(0, nv, body, 0)
        pltpu.sync_copy(dxv, ox_hbm.at[pl.ds(base, per)])
        pltpu.sync_copy(dyv, oy_hbm.at[pl.ds(base, per)])

    return k(xt, yt, jloc, iloc, maskf)


def _sc_posj(xt, yt, jloc, rpb, epb):
    E = jloc.shape[0]
    per = E // NW
    nv = per // 16
    mesh = plsc.VectorSubcoreMesh(core_axis_name="c", subcore_axis_name="s")

    @functools.partial(
        pl.kernel, mesh=mesh,
        compiler_params=pltpu.CompilerParams(needs_layout_passes=False),
        out_type=(jax.ShapeDtypeStruct((E,), jnp.float32),
                  jax.ShapeDtypeStruct((E,), jnp.float32)),
        scratch_types=[
            pltpu.VMEM((rpb,), jnp.float32),
            pltpu.VMEM((rpb,), jnp.float32),
            pltpu.VMEM((per,), jnp.int32),
            pltpu.VMEM((per,), jnp.float32),
            pltpu.VMEM((per,), jnp.float32),
        ],
    )
    def k(xt_hbm, yt_hbm, j_hbm, ox_hbm, oy_hbm, xv, yv, jv, pxv, pyv):
        wid = lax.axis_index("s") * 2 + lax.axis_index("c")
        base = wid * per
        b = (wid * per) // epb
        pltpu.sync_copy(xt_hbm.at[pl.ds(b * rpb, rpb)], xv)
        pltpu.sync_copy(yt_hbm.at[pl.ds(b * rpb, rpb)], yv)
        pltpu.sync_copy(j_hbm.at[pl.ds(base, per)], jv)

        def body(v, c):
            off = pl.multiple_of(v * 16, 16)
            j = jv[pl.ds(off, 16)]
            pxv[pl.ds(off, 16)] = plsc.load_gather(xv, [j])
            pyv[pl.ds(off, 16)] = plsc.load_gather(yv, [j])
            return c

        lax.fori_loop(0, nv, body, 0)
        pltpu.sync_copy(pxv, ox_hbm.at[pl.ds(base, per)])
        pltpu.sync_copy(pyv, oy_hbm.at[pl.ds(base, per)])

    return k(xt, yt, jloc)


def _rep(x, k):
    blk, w = x.shape
    return jnp.broadcast_to(x[:, None, :], (blk, k, w)).reshape(blk * k, w)


def _embed_body(pf_ref, ke_ref, evW_ref, evb_ref, eaW_ref, eab_ref,
                embW_ref, embb_ref, w1b_ref, h_ref, b_ref):
    pf = pf_ref[...]
    embW = embW_ref[...]
    spd = jnp.sqrt(pf[:, 2:3] ** 2 + pf[:, 3:4] ** 2)
    acc = jnp.sqrt(pf[:, 4:5] ** 2 + pf[:, 5:6] ** 2)
    cv = evW_ref[...] @ embW[0:8]
    ca = eaW_ref[...] @ embW[8:16]
    c0 = evb_ref[...] @ embW[0:8] + eab_ref[...] @ embW[8:16] + embb_ref[...]
    h = spd * cv + acc * ca + ke_ref[...] @ embW[16:19] + c0
    h_ref[...] = h
    b_ref[...] = h @ w1b_ref[...]


def _embed_call(pf, ke, pr):
    grid = (BN // BLK,)
    full = lambda s: pl.BlockSpec(s, lambda g: (0, 0))
    return pl.pallas_call(
        _embed_body,
        grid=grid,
        in_specs=[
            pl.BlockSpec((BLK, 6), lambda g: (g, 0)),
            pl.BlockSpec((BLK, 3), lambda g: (g, 0)),
            full((1, 8)), full((1, 8)), full((1, 8)), full((1, 8)),
            full((19, HID)), full((1, HID)), full((HID, HID)),
        ],
        out_specs=[
            pl.BlockSpec((BLK, HID), lambda g: (g, 0)),
            pl.BlockSpec((BLK, HID), lambda g: (g, 0)),
        ],
        out_shape=[
            jax.ShapeDtypeStruct((BN, HID), jnp.float32),
            jax.ShapeDtypeStruct((BN, HID), jnp.float32),
        ],
    )(pf, ke,
      pr["enc_v_W"].reshape(1, 8), pr["enc_v_b"].reshape(1, 8),
      pr["enc_a_W"].reshape(1, 8), pr["enc_a_b"].reshape(1, 8),
      pr["emb_W"], pr["emb_b"].reshape(1, HID),
      pr["gnn0_W1b"])


def _edge_math(h, st, Bj, dxm, dym, mask128, maskD, w, kk):
    eb = BLK * kk
    r = eb // 128
    d = jnp.sqrt(dxm * dxm + dym * dym)
    A = h @ w["W1a"] + w["feb1"]
    base = _rep(A, kk)
    if Bj is not None:
        base = base + Bj
    pre1 = base.reshape(r, 128, HID) + d[:, :, None] * w["w1d"].reshape(1, 1, HID)
    u = _silu(pre1).reshape(eb, HID)
    m3 = (_silu(u @ w["feW2"] + w["feb2"]).reshape(r, 128, HID)
          * mask128[:, :, None])
    m = m3.reshape(eb, HID)
    t = _silu(m @ w["fxW1"] + w["fxb1"])
    s = jnp.sum(t.reshape(r, 128, HID) * w["fxW2r"].reshape(1, 1, HID),
                axis=2) + w["fxb2"]
    r3 = jnp.concatenate(
        [m3, (dxm * s)[:, :, None], (dym * s)[:, :, None]], axis=2)
    rr = jnp.sum(r3.reshape(BLK, kk, HID + 2), axis=1)
    m_i = rr[:, 0:HID]
    aggx = rr[:, HID:HID + 1]
    aggy = rr[:, HID + 1:HID + 2]
    nn = jnp.sum(maskD, axis=1, keepdims=True)
    inv = 1.0 / (nn + 1e-6)
    fa = _silu(h @ w["faW1"] + w["fab1"]) @ w["faW2c"] + w["fab2"]
    ax = fa * st[:, 4:5] + aggx * inv
    ay = fa * st[:, 5:6] + aggy * inv
    vx = st[:, 2:3] + ax
    vy = st[:, 3:4] + ay
    xx = st[:, 0:1] + vx
    xy = st[:, 1:2] + vy
    h_new = h + _silu(h @ w["fhW1h"] + m_i @ w["fhW1m"] + w["fhb1"]) @ w["fhW2"] + w["fhb2"]
    st_new = jnp.concatenate([xx, xy, vx, vy, st[:, 4:8]], axis=1)
    return h_new, st_new


_WNAMES = ["W1a", "w1d", "feb1", "feW2", "feb2", "fxW1", "fxb1", "fxW2r",
           "fxb2", "faW1", "fab1", "faW2c", "fab2", "fhW1h", "fhW1m",
           "fhb1", "fhW2", "fhb2"]


def _edge_body(h_ref, st_ref, bj_ref, dx_ref, dy_ref, mp_ref, md_ref, *w_refs):
    (w1bn_ref, *wl), outs = (w_refs[:-5], w_refs[-5:])
    hn_ref, stn_ref, bn_ref, px_ref, py_ref = outs
    w = {n: r[...] for n, r in zip(_WNAMES, wl)}
    h = h_ref[...]
    st = st_ref[...]
    h_new, st_new = _edge_math(h, st, bj_ref[...], dx_ref[...], dy_ref[...],
                               mp_ref[...], md_ref[...], w, K)
    hn_ref[...] = h_new
    stn_ref[...] = st_new
    bn_ref[...] = h_new @ w1bn_ref[...]
    px_ref[...] = st_new[:, 0:1]
    py_ref[...] = st_new[:, 1:2]


def _edge_call(h, st, bj, dxm, dym, maskP, maskD, wl, w1bn):
    grid = (BN // BLK,)
    full = lambda s: pl.BlockSpec(s, lambda g: tuple(0 for _ in s))
    wspecs = [full(w1bn.shape)] + [full(wl[n].shape) for n in _WNAMES]
    rp = (BLK * K) // 128
    return pl.pallas_call(
        _edge_body,
        grid=grid,
        in_specs=[
            pl.BlockSpec((BLK, HID), lambda g: (g, 0)),
            pl.BlockSpec((BLK, 8), lambda g: (g, 0)),
            pl.BlockSpec((BLK * K, HID), lambda g: (g, 0)),
            pl.BlockSpec((rp, 128), lambda g: (g, 0)),
            pl.BlockSpec((rp, 128), lambda g: (g, 0)),
            pl.BlockSpec((rp, 128), lambda g: (g, 0)),
            pl.BlockSpec((BLK, K), lambda g: (g, 0)),
        ] + wspecs,
        out_specs=[
            pl.BlockSpec((BLK, HID), lambda g: (g, 0)),
            pl.BlockSpec((BLK, 8), lambda g: (g, 0)),
            pl.BlockSpec((BLK, HID), lambda g: (g, 0)),
            pl.BlockSpec((BLK, 1), lambda g: (g, 0)),
            pl.BlockSpec((BLK, 1), lambda g: (g, 0)),
        ],
        out_shape=[
            jax.ShapeDtypeStruct((BN, HID), jnp.float32),
            jax.ShapeDtypeStruct((BN, 8), jnp.float32),
            jax.ShapeDtypeStruct((BN, HID), jnp.float32),
            jax.ShapeDtypeStruct((BN, 1), jnp.float32),
            jax.ShapeDtypeStruct((BN, 1), jnp.float32),
        ],
    )(h, st, bj, dxm, dym, maskP, maskD, w1bn, *[wl[n] for n in _WNAMES])


def _obs_body(h_ref, st_ref, ox_ref, oy_ref, mp_ref, md_ref, *w_refs):
    wl3, (hn_ref,) = w_refs[:-1], w_refs[-1:]
    h = h_ref[...]
    st = st_ref[...]
    rp = (BLK * KO) // 128
    mask128 = mp_ref[...]
    maskD = md_ref[...]
    opx = ox_ref[...]
    opy = oy_ref[...]
    for l in range(3):
        w = {n: r[l] for n, r in zip(_WNAMES, wl3)}
        pix = _rep(st[:, 0:1], KO).reshape(rp, 128)
        piy = _rep(st[:, 1:2], KO).reshape(rp, 128)
        dxm = (opx - pix) * mask128
        dym = (opy - piy) * mask128
        h, st = _edge_math(h, st, None, dxm, dym, mask128, maskD, w, KO)
    hn_ref[...] = h


def _obs_call(h, st, opx, opy, maskP, maskD, wl3):
    grid = (BN // BLK,)
    full = lambda s: pl.BlockSpec(s, lambda g: tuple(0 for _ in s))
    wspecs = [full(wl3[n].shape) for n in _WNAMES]
    rp = (BLK * KO) // 128
    return pl.pallas_call(
        _obs_body,
        grid=grid,
        in_specs=[
            pl.BlockSpec((BLK, HID), lambda g: (g, 0)),
            pl.BlockSpec((BLK, 8), lambda g: (g, 0)),
            pl.BlockSpec((rp, 128), lambda g: (g, 0)),
            pl.BlockSpec((rp, 128), lambda g: (g, 0)),
            pl.BlockSpec((rp, 128), lambda g: (g, 0)),
            pl.BlockSpec((BLK, KO), lambda g: (g, 0)),
        ] + wspecs,
        out_specs=pl.BlockSpec((BLK, HID), lambda g: (g, 0)),
        out_shape=jax.ShapeDtypeStruct((BN, HID), jnp.float32),
    )(h, st, opx, opy, maskP, maskD, *[wl3[n] for n in _WNAMES])


def _prep_layer(p, has_neigh):
    W1 = p["fe_W1"]
    w = {
        "W1a": W1[:HID],
        "w1d": W1[-1:],
        "feb1": p["fe_b1"].reshape(1, HID),
        "feW2": p["fe_W2"],
        "feb2": p["fe_b2"].reshape(1, HID),
        "fxW1": p["fx_W1"],
        "fxb1": p["fx_b1"].reshape(1, HID),
        "fxW2r": p["fx_W2"].reshape(1, HID),
        "fxb2": p["fx_b2"].reshape(1, 1),
        "faW1": p["fa_W1"],
        "fab1": p["fa_b1"].reshape(1, HID),
        "faW2c": p["fa_W2"].reshape(HID, 1),
        "fab2": p["fa_b2"].reshape(1, 1),
        "fhW1h": p["fh_W1"][:HID],
        "fhW1m": p["fh_W1"][HID:2 * HID],
        "fhb1": p["fh_b1"].reshape(1, HID),
        "fhW2": p["fh_W2"],
        "fhb2": p["fh_b2"].reshape(1, HID),
    }
    if has_neigh:
        w["W1b"] = W1[HID:2 * HID]
    return w


def kernel(ped_features, neigh_mask, neigh_index, obs_features,
           neigh_mask_obs, neigh_index_obs, k_emb, params):
    pf = ped_features.reshape(BN, 6)
    ke = k_emb.reshape(BN, 3)
    st0 = jnp.pad(pf, ((0, 0), (0, 2)))

    offs = (jnp.arange(BS, dtype=jnp.int32) * N)[:, None, None]
    gidx = (neigh_index.astype(jnp.int32) + offs).reshape(BN * K)
    jloc = neigh_index.astype(jnp.int32).reshape(BN * K)
    iloc = jnp.repeat(jnp.arange(BN, dtype=jnp.int32) % N, K)
    ojloc = neigh_index_obs.astype(jnp.int32).reshape(BN * KO)
    maskf = neigh_mask.astype(jnp.float32)
    maskE = maskf.reshape(BN * K)
    maskP = maskf.reshape(BN * K // 128, 128)
    maskD = maskf.reshape(BN, K)
    maskfo = neigh_mask_obs.astype(jnp.float32)
    maskPo = maskfo.reshape(BN * KO // 128, 128)
    maskDo = maskfo.reshape(BN, KO)

    gl = [_prep_layer(p, True) for p in params["gnn"]]
    ol = [_prep_layer(p, False) for p in params["obs"]]
    ol3 = {n: jnp.stack([w[n] for w in ol]) for n in _WNAMES}

    pr = dict(params)
    pr["gnn0_W1b"] = gl[0]["W1b"]
    h1, B1 = _embed_call(pf, ke, pr)

    obs_x = obs_features[..., 0].reshape(BS * M)
    obs_y = obs_features[..., 1].reshape(BS * M)
    opx, opy = _sc_posj(obs_x, obs_y, ojloc, M, N * KO)
    opx = opx.reshape(BN * KO // 128, 128)
    opy = opy.reshape(BN * KO // 128, 128)

    h_obs = _obs_call(h1, st0, opx, opy, maskPo, maskDo, ol3)

    h, st, B = h1, st0, B1
    px = pf[:, 0:1]
    py = pf[:, 1:2]
    zW = jnp.zeros((HID, HID), jnp.float32)
    for li in range(3):
        dxm, dym = _sc_dxy(px.reshape(BN), py.reshape(BN), jloc, iloc,
                           maskE, N, N * K)
        bj = _sc_gather_rows(B, gidx, HID)
        w1bn = gl[li + 1]["W1b"] if li < 2 else zW
        h, st, B, px, py = _edge_call(
            h, st, bj,
            dxm.reshape(BN * K // 128, 128), dym.reshape(BN * K // 128, 128),
            maskP, maskD, gl[li], w1bn)

    return (h.reshape(BS, N, HID), h_obs.reshape(BS, N, HID))


# speedup vs baseline: 1.0497x; 1.0248x over previous
import functools

import jax
import jax.numpy as jnp
from jax import lax
from jax.experimental import pallas as pl
from jax.experimental.pallas import tpu as pltpu
from jax.experimental.pallas import tpu_sc as plsc

HID = 64
BS, N, K, M, KO = 8, 1024, 32, 256, 16
BN = BS * N
BLK = 256
NW = 32
GC = 128


def _silu(x):
    return x * jax.nn.sigmoid(x)


def _sc_gather_rows(table, idx, D):
    E = idx.shape[0]
    per = E // NW
    nch = per // GC
    idx3 = idx.reshape(NW, nch, GC)
    mesh = plsc.VectorSubcoreMesh(core_axis_name="c", subcore_axis_name="s")

    @functools.partial(
        pl.kernel, mesh=mesh,
        compiler_params=pltpu.CompilerParams(use_tc_tiling_on_sc=False),
        out_type=jax.ShapeDtypeStruct((E, D), jnp.float32),
        scratch_types=[
            pltpu.VMEM((nch, GC), jnp.int32),
            pltpu.VMEM((4, GC, D), jnp.float32),
            pltpu.SemaphoreType.DMA,
            pltpu.SemaphoreType.DMA,
            pltpu.SemaphoreType.DMA,
            pltpu.SemaphoreType.DMA,
            pltpu.SemaphoreType.DMA,
            pltpu.SemaphoreType.DMA,
        ],
    )
    def k(table_hbm, idx_hbm, out_hbm, idx_v, rows_v,
          g0, g1, s0, s1, s2, s3):
        wid = lax.axis_index("s") * 2 + lax.axis_index("c")
        base = wid * per
        pltpu.sync_copy(idx_hbm.at[wid], idx_v)
        gsems = [g0, g1]
        ssems = [s0, s1, s2, s3]
        gathers = [None] * 4
        stores = [None] * 4
        for g in range(nch + 1):
            if g < nch:
                b = g % 4
                if stores[b] is not None:
                    stores[b].wait()
                gathers[b] = pltpu.async_copy(
                    table_hbm.at[idx_v.at[g]], rows_v.at[b], gsems[g % 2])
            if g >= 1:
                pb = (g - 1) % 4
                gathers[pb].wait()
                stores[pb] = pltpu.async_copy(
                    rows_v.at[pb],
                    out_hbm.at[pl.ds(base + (g - 1) * GC, GC)], ssems[pb])
        for s in stores:
            if s is not None:
                s.wait()

    return k(table, idx3)


def _sc_dxy(xt, yt, jloc, iloc, maskf, rpb, epb):
    E = jloc.shape[0]
    per = E // NW
    nv = per // 16
    mesh = plsc.VectorSubcoreMesh(core_axis_name="c", subcore_axis_name="s")

    @functools.partial(
        pl.kernel, mesh=mesh,
        compiler_params=pltpu.CompilerParams(needs_layout_passes=False),
        out_type=(jax.ShapeDtypeStruct((E,), jnp.float32),
                  jax.ShapeDtypeStruct((E,), jnp.float32)),
        scratch_types=[
            pltpu.VMEM((rpb,), jnp.float32),
            pltpu.VMEM((rpb,), jnp.float32),
            pltpu.VMEM((per,), jnp.int32),
            pltpu.VMEM((per,), jnp.int32),
            pltpu.VMEM((per,), jnp.float32),
            pltpu.VMEM((per,), jnp.float32),
            pltpu.VMEM((per,), jnp.float32),
        ],
    )
    def k(xt_hbm, yt_hbm, j_hbm, i_hbm, m_hbm, ox_hbm, oy_hbm,
          xv, yv, jv, iv, mv, dxv, dyv):
        wid = lax.axis_index("s") * 2 + lax.axis_index("c")
        base = wid * per
        b = (wid * per) // epb
        pltpu.sync_copy(xt_hbm.at[pl.ds(b * rpb, rpb)], xv)
        pltpu.sync_copy(yt_hbm.at[pl.ds(b * rpb, rpb)], yv)
        pltpu.sync_copy(j_hbm.at[pl.ds(base, per)], jv)
        pltpu.sync_copy(i_hbm.at[pl.ds(base, per)], iv)
        pltpu.sync_copy(m_hbm.at[pl.ds(base, per)], mv)

        def body(v, c):
            off = pl.multiple_of(v * 16, 16)
            j = jv[pl.ds(off, 16)]
            i = iv[pl.ds(off, 16)]
            mk = mv[pl.ds(off, 16)]
            dxv[pl.ds(off, 16)] = (
                plsc.load_gather(xv, [j]) - plsc.load_gather(xv, [i])) * mk
            dyv[pl.ds(off, 16)] = (
                plsc.load_gather(yv, [j]) - plsc.load_gather(yv, [i])) * mk
            return c

        lax.fori_loop(0, nv, body, 0)
        pltpu.sync_copy(dxv, ox_hbm.at[pl.ds(base, per)])
        pltpu.sync_copy(dyv, oy_hbm.at[pl.ds(base, per)])

    return k(xt, yt, jloc, iloc, maskf)


def _sc_posj(xt, yt, jloc, rpb, epb):
    E = jloc.shape[0]
    per = E // NW
    nv = per // 16
    mesh = plsc.VectorSubcoreMesh(core_axis_name="c", subcore_axis_name="s")

    @functools.partial(
        pl.kernel, mesh=mesh,
        compiler_params=pltpu.CompilerParams(needs_layout_passes=False),
        out_type=(jax.ShapeDtypeStruct((E,), jnp.float32),
                  jax.ShapeDtypeStruct((E,), jnp.float32)),
        scratch_types=[
            pltpu.VMEM((rpb,), jnp.float32),
            pltpu.VMEM((rpb,), jnp.float32),
            pltpu.VMEM((per,), jnp.int32),
            pltpu.VMEM((per,), jnp.float32),
            pltpu.VMEM((per,), jnp.float32),
        ],
    )
    def k(xt_hbm, yt_hbm, j_hbm, ox_hbm, oy_hbm, xv, yv, jv, pxv, pyv):
        wid = lax.axis_index("s") * 2 + lax.axis_index("c")
        base = wid * per
        b = (wid * per) // epb
        pltpu.sync_copy(xt_hbm.at[pl.ds(b * rpb, rpb)], xv)
        pltpu.sync_copy(yt_hbm.at[pl.ds(b * rpb, rpb)], yv)
        pltpu.sync_copy(j_hbm.at[pl.ds(base, per)], jv)

        def body(v, c):
            off = pl.multiple_of(v * 16, 16)
            j = jv[pl.ds(off, 16)]
            pxv[pl.ds(off, 16)] = plsc.load_gather(xv, [j])
            pyv[pl.ds(off, 16)] = plsc.load_gather(yv, [j])
            return c

        lax.fori_loop(0, nv, body, 0)
        pltpu.sync_copy(pxv, ox_hbm.at[pl.ds(base, per)])
        pltpu.sync_copy(pyv, oy_hbm.at[pl.ds(base, per)])

    return k(xt, yt, jloc)


def _rep(x, k):
    blk, w = x.shape
    return jnp.broadcast_to(x[:, None, :], (blk, k, w)).reshape(blk * k, w)


def _embed_body(pf_ref, ke_ref, evW_ref, evb_ref, eaW_ref, eab_ref,
                embW_ref, embb_ref, w1b_ref, h_ref, b_ref):
    pf = pf_ref[...]
    embW = embW_ref[...]
    spd = jnp.sqrt(pf[:, 2:3] ** 2 + pf[:, 3:4] ** 2)
    acc = jnp.sqrt(pf[:, 4:5] ** 2 + pf[:, 5:6] ** 2)
    cv = evW_ref[...] @ embW[0:8]
    ca = eaW_ref[...] @ embW[8:16]
    c0 = evb_ref[...] @ embW[0:8] + eab_ref[...] @ embW[8:16] + embb_ref[...]
    h = spd * cv + acc * ca + ke_ref[...] @ embW[16:19] + c0
    h_ref[...] = h
    b_ref[...] = h @ w1b_ref[...]


def _embed_call(pf, ke, pr):
    grid = (BN // BLK,)
    full = lambda s: pl.BlockSpec(s, lambda g: (0, 0))
    return pl.pallas_call(
        _embed_body,
        grid=grid,
        in_specs=[
            pl.BlockSpec((BLK, 6), lambda g: (g, 0)),
            pl.BlockSpec((BLK, 3), lambda g: (g, 0)),
            full((1, 8)), full((1, 8)), full((1, 8)), full((1, 8)),
            full((19, HID)), full((1, HID)), full((HID, HID)),
        ],
        out_specs=[
            pl.BlockSpec((BLK, HID), lambda g: (g, 0)),
            pl.BlockSpec((BLK, HID), lambda g: (g, 0)),
        ],
        out_shape=[
            jax.ShapeDtypeStruct((BN, HID), jnp.float32),
            jax.ShapeDtypeStruct((BN, HID), jnp.float32),
        ],
    )(pf, ke,
      pr["enc_v_W"].reshape(1, 8), pr["enc_v_b"].reshape(1, 8),
      pr["enc_a_W"].reshape(1, 8), pr["enc_a_b"].reshape(1, 8),
      pr["emb_W"], pr["emb_b"].reshape(1, HID),
      pr["gnn0_W1b"])


def _edge_math(h, st, Bj, dxm, dym, mask128, maskD, w, kk):
    eb = BLK * kk
    r = eb // 128
    d = jnp.sqrt(dxm * dxm + dym * dym)
    A = h @ w["W1a"] + w["feb1"]
    base = _rep(A, kk)
    if Bj is not None:
        base = base + Bj
    pre1 = base.reshape(r, 128, HID) + d[:, :, None] * w["w1d"].reshape(1, 1, HID)
    u = _silu(pre1).reshape(eb, HID)
    m3 = (_silu(u @ w["feW2"] + w["feb2"]).reshape(r, 128, HID)
          * mask128[:, :, None])
    m = m3.reshape(eb, HID)
    t = _silu(m @ w["fxW1"] + w["fxb1"])
    s = (t @ w["fxW2c"]).reshape(r, 128) + w["fxb2"]
    r3 = jnp.concatenate(
        [m3, (dxm * s)[:, :, None], (dym * s)[:, :, None]], axis=2)
    rr = jnp.sum(r3.reshape(BLK, kk, HID + 2), axis=1)
    m_i = rr[:, 0:HID]
    aggx = rr[:, HID:HID + 1]
    aggy = rr[:, HID + 1:HID + 2]
    nn = jnp.sum(maskD, axis=1, keepdims=True)
    inv = 1.0 / (nn + 1e-6)
    fa = _silu(h @ w["faW1"] + w["fab1"]) @ w["faW2c"] + w["fab2"]
    ax = fa * st[:, 4:5] + aggx * inv
    ay = fa * st[:, 5:6] + aggy * inv
    vx = st[:, 2:3] + ax
    vy = st[:, 3:4] + ay
    xx = st[:, 0:1] + vx
    xy = st[:, 1:2] + vy
    h_new = h + _silu(h @ w["fhW1h"] + m_i @ w["fhW1m"] + w["fhb1"]) @ w["fhW2"] + w["fhb2"]
    st_new = jnp.concatenate([xx, xy, vx, vy, st[:, 4:8]], axis=1)
    return h_new, st_new


_WNAMES = ["W1a", "w1d", "feb1", "feW2", "feb2", "fxW1", "fxb1", "fxW2c",
           "fxb2", "faW1", "fab1", "faW2c", "fab2", "fhW1h", "fhW1m",
           "fhb1", "fhW2", "fhb2"]


def _edge_body(h_ref, st_ref, bj_ref, dx_ref, dy_ref, mp_ref, md_ref, *w_refs):
    (w1bn_ref, *wl), outs = (w_refs[:-5], w_refs[-5:])
    hn_ref, stn_ref, bn_ref, px_ref, py_ref = outs
    w = {n: r[...] for n, r in zip(_WNAMES, wl)}
    h = h_ref[...]
    st = st_ref[...]
    h_new, st_new = _edge_math(h, st, bj_ref[...], dx_ref[...], dy_ref[...],
                               mp_ref[...], md_ref[...], w, K)
    hn_ref[...] = h_new
    stn_ref[...] = st_new
    bn_ref[...] = h_new @ w1bn_ref[...]
    px_ref[...] = st_new[:, 0:1]
    py_ref[...] = st_new[:, 1:2]


def _edge_call(h, st, bj, dxm, dym, maskP, maskD, wl, w1bn):
    grid = (BN // BLK,)
    full = lambda s: pl.BlockSpec(s, lambda g: tuple(0 for _ in s))
    wspecs = [full(w1bn.shape)] + [full(wl[n].shape) for n in _WNAMES]
    rp = (BLK * K) // 128
    return pl.pallas_call(
        _edge_body,
        grid=grid,
        in_specs=[
            pl.BlockSpec((BLK, HID), lambda g: (g, 0)),
            pl.BlockSpec((BLK, 8), lambda g: (g, 0)),
            pl.BlockSpec((BLK * K, HID), lambda g: (g, 0)),
            pl.BlockSpec((rp, 128), lambda g: (g, 0)),
            pl.BlockSpec((rp, 128), lambda g: (g, 0)),
            pl.BlockSpec((rp, 128), lambda g: (g, 0)),
            pl.BlockSpec((BLK, K), lambda g: (g, 0)),
        ] + wspecs,
        out_specs=[
            pl.BlockSpec((BLK, HID), lambda g: (g, 0)),
            pl.BlockSpec((BLK, 8), lambda g: (g, 0)),
            pl.BlockSpec((BLK, HID), lambda g: (g, 0)),
            pl.BlockSpec((BLK, 1), lambda g: (g, 0)),
            pl.BlockSpec((BLK, 1), lambda g: (g, 0)),
        ],
        out_shape=[
            jax.ShapeDtypeStruct((BN, HID), jnp.float32),
            jax.ShapeDtypeStruct((BN, 8), jnp.float32),
            jax.ShapeDtypeStruct((BN, HID), jnp.float32),
            jax.ShapeDtypeStruct((BN, 1), jnp.float32),
            jax.ShapeDtypeStruct((BN, 1), jnp.float32),
        ],
    )(h, st, bj, dxm, dym, maskP, maskD, w1bn, *[wl[n] for n in _WNAMES])


def _obs_body(h_ref, st_ref, ox_ref, oy_ref, mp_ref, md_ref, rsel_ref,
              sprd_ref, *w_refs):
    wl3, (hn_ref,) = w_refs[:-1], w_refs[-1:]
    h = h_ref[...]
    st = st_ref[...]
    rp = (BLK * KO) // 128
    mask128 = mp_ref[...]
    maskD = md_ref[...]
    opx = ox_ref[...]
    opy = oy_ref[...]
    rsel = rsel_ref[...]
    sprd = sprd_ref[...]
    for l in range(3):
        w = {n: r[l] for n, r in zip(_WNAMES, wl3)}
        pt = jnp.transpose(st[:, 0:2])
        pix = (jnp.broadcast_to(pt[0:1, :], (rp, BLK)) * rsel) @ sprd
        piy = (jnp.broadcast_to(pt[1:2, :], (rp, BLK)) * rsel) @ sprd
        dxm = (opx - pix) * mask128
        dym = (opy - piy) * mask128
        h, st = _edge_math(h, st, None, dxm, dym, mask128, maskD, w, KO)
    hn_ref[...] = h


def _obs_call(h, st, opx, opy, maskP, maskD, rsel, sprd, wl3):
    grid = (BN // BLK,)
    full = lambda s: pl.BlockSpec(s, lambda g: tuple(0 for _ in s))
    wspecs = [full(wl3[n].shape) for n in _WNAMES]
    rp = (BLK * KO) // 128
    return pl.pallas_call(
        _obs_body,
        grid=grid,
        in_specs=[
            pl.BlockSpec((BLK, HID), lambda g: (g, 0)),
            pl.BlockSpec((BLK, 8), lambda g: (g, 0)),
            pl.BlockSpec((rp, 128), lambda g: (g, 0)),
            pl.BlockSpec((rp, 128), lambda g: (g, 0)),
            pl.BlockSpec((rp, 128), lambda g: (g, 0)),
            pl.BlockSpec((BLK, KO), lambda g: (g, 0)),
            full((rp, BLK)), full((BLK, 128)),
        ] + wspecs,
        out_specs=pl.BlockSpec((BLK, HID), lambda g: (g, 0)),
        out_shape=jax.ShapeDtypeStruct((BN, HID), jnp.float32),
    )(h, st, opx, opy, maskP, maskD, rsel, sprd, *[wl3[n] for n in _WNAMES])


def _prep_layer(p, has_neigh):
    W1 = p["fe_W1"]
    w = {
        "W1a": W1[:HID],
        "w1d": W1[-1:],
        "feb1": p["fe_b1"].reshape(1, HID),
        "feW2": p["fe_W2"],
        "feb2": p["fe_b2"].reshape(1, HID),
        "fxW1": p["fx_W1"],
        "fxb1": p["fx_b1"].reshape(1, HID),
        "fxW2c": p["fx_W2"].reshape(HID, 1),
        "fxb2": p["fx_b2"].reshape(1, 1),
        "faW1": p["fa_W1"],
        "fab1": p["fa_b1"].reshape(1, HID),
        "faW2c": p["fa_W2"].reshape(HID, 1),
        "fab2": p["fa_b2"].reshape(1, 1),
        "fhW1h": p["fh_W1"][:HID],
        "fhW1m": p["fh_W1"][HID:2 * HID],
        "fhb1": p["fh_b1"].reshape(1, HID),
        "fhW2": p["fh_W2"],
        "fhb2": p["fh_b2"].reshape(1, HID),
    }
    if has_neigh:
        w["W1b"] = W1[HID:2 * HID]
    return w


def kernel(ped_features, neigh_mask, neigh_index, obs_features,
           neigh_mask_obs, neigh_index_obs, k_emb, params):
    pf = ped_features.reshape(BN, 6)
    ke = k_emb.reshape(BN, 3)
    st0 = jnp.pad(pf, ((0, 0), (0, 2)))

    offs = (jnp.arange(BS, dtype=jnp.int32) * N)[:, None, None]
    gidx = (neigh_index.astype(jnp.int32) + offs).reshape(BN * K)
    jloc = neigh_index.astype(jnp.int32).reshape(BN * K)
    iloc = jnp.repeat(jnp.arange(BN, dtype=jnp.int32) % N, K)
    ojloc = neigh_index_obs.astype(jnp.int32).reshape(BN * KO)
    maskf = neigh_mask.astype(jnp.float32)
    maskE = maskf.reshape(BN * K)
    maskP = maskf.reshape(BN * K // 128, 128)
    maskD = maskf.reshape(BN, K)
    maskfo = neigh_mask_obs.astype(jnp.float32)
    maskPo = maskfo.reshape(BN * KO // 128, 128)
    maskDo = maskfo.reshape(BN, KO)

    gl = [_prep_layer(p, True) for p in params["gnn"]]
    ol = [_prep_layer(p, False) for p in params["obs"]]
    ol3 = {n: jnp.stack([w[n] for w in ol]) for n in _WNAMES}

    pr = dict(params)
    pr["gnn0_W1b"] = gl[0]["W1b"]
    h1, B1 = _embed_call(pf, ke, pr)

    obs_x = obs_features[..., 0].reshape(BS * M)
    obs_y = obs_features[..., 1].reshape(BS * M)
    opx, opy = _sc_posj(obs_x, obs_y, ojloc, M, N * KO)
    opx = opx.reshape(BN * KO // 128, 128)
    opy = opy.reshape(BN * KO // 128, 128)

    rp_o = (BLK * KO) // 128
    npr = 128 // KO
    rsel = (jnp.arange(BLK)[None, :] // npr
            == jnp.arange(rp_o)[:, None]).astype(jnp.float32)
    sprd = (jnp.arange(128)[None, :] // KO
            == (jnp.arange(BLK)[:, None] % npr)).astype(jnp.float32)
    h_obs = _obs_call(h1, st0, opx, opy, maskPo, maskDo, rsel, sprd, ol3)

    h, st, B = h1, st0, B1
    px = pf[:, 0:1]
    py = pf[:, 1:2]
    zW = jnp.zeros((HID, HID), jnp.float32)
    for li in range(3):
        dxm, dym = _sc_dxy(px.reshape(BN), py.reshape(BN), jloc, iloc,
                           maskE, N, N * K)
        bj = _sc_gather_rows(B, gidx, HID)
        w1bn = gl[li + 1]["W1b"] if li < 2 else zW
        h, st, B, px, py = _edge_call(
            h, st, bj,
            dxm.reshape(BN * K // 128, 128), dym.reshape(BN * K // 128, 128),
            maskP, maskD, gl[li], w1bn)

    return (h.reshape(BS, N, HID), h_obs.reshape(BS, N, HID))
